# baseline (speedup 1.0000x reference)
import functools

import jax
import jax.numpy as jnp
from jax import lax
from jax.experimental import pallas as pl
from jax.experimental.pallas import tpu as pltpu
from jax.experimental.pallas import tpu_sc as plsc

H = 128
EH = 16
E = 320000
N = 10000
G = 2500
EDGE_SCALE = 0.1

NC = 2
NS = 16
NW = NC * NS
EPW = E // NW
CH = 80
NCH = EPW // CH
RPT = 632
RPT_LAST = N - (NS - 1) * RPT

EB = 2560
EGRID = E // EB
NB = 2000
NGRID = N // NB
FB = 400
FGRID = N // FB

PS1 = 64
PS2 = 16
PDP = 16


def _gelu(x):
    return 0.5 * x * (1.0 + lax.erf(x * 0.7071067811865476))


def _ln(x, g, b, eps=1e-5):
    mu = jnp.mean(x, axis=-1, keepdims=True)
    var = jnp.var(x, axis=-1, keepdims=True)
    return (x - mu) / jnp.sqrt(var + eps) * g + b


def _full(x):
    return pl.BlockSpec(x.shape, lambda *_: (0,) * x.ndim)


def _edge_in_body(ea_ref, w_ref, b_ref, g_ref, bb_ref, out_ref):
    x = ea_ref[...] @ w_ref[...] + b_ref[...]
    out_ref[...] = _ln(_gelu(x), g_ref[...], bb_ref[...])


def _edge_in(edge_attr, W, b, g, bb):
    return pl.pallas_call(
        _edge_in_body,
        grid=(EGRID,),
        in_specs=[pl.BlockSpec((EB, EH), lambda i: (i, 0)),
                  _full(W), _full(b), _full(g), _full(bb)],
        out_specs=pl.BlockSpec((EB, EH), lambda i: (i, 0)),
        out_shape=jax.ShapeDtypeStruct((E, EH), jnp.float32),
    )(edge_attr, W, b, g, bb)


def _sc_gather(p_src1, p_src2, p_dst, src2, dst2):
    mesh = plsc.VectorSubcoreMesh(core_axis_name="c", subcore_axis_name="s",
                                  num_cores=NC, num_subcores=NS)

    @functools.partial(
        pl.kernel,
        out_type=(jax.ShapeDtypeStruct((E, PS1), jnp.int32),
                  jax.ShapeDtypeStruct((E, PS2), jnp.int32),
                  jax.ShapeDtypeStruct((E, PDP), jnp.int32)),
        mesh=mesh,
        scratch_types=[
            pltpu.VMEM((EPW,), jnp.int32),
            pltpu.VMEM((EPW,), jnp.int32),
            pltpu.VMEM((CH, PS1), jnp.int32),
            pltpu.VMEM((CH, PS2), jnp.int32),
            pltpu.VMEM((CH, PDP), jnp.int32),
            pltpu.SemaphoreType.DMA,
            pltpu.SemaphoreType.DMA,
            pltpu.SemaphoreType.DMA,
        ],
        compiler_params=pltpu.CompilerParams(use_tc_tiling_on_sc=False),
    )
    def k(p1_hbm, p2_hbm, pd_hbm, src_hbm, dst_hbm, g1_out, g2_out, gd_out,
          sidx, didx, r1, r2, rd, sem1, sem2, semd):
        wid = lax.axis_index("s") * NC + lax.axis_index("c")
        base = wid * EPW
        pltpu.sync_copy(src_hbm.at[wid], sidx)
        pltpu.sync_copy(dst_hbm.at[wid], didx)

        def body(j, carry):
            off = j * CH
            sl = sidx.at[pl.ds(off, CH)]
            c1 = pltpu.async_copy(p1_hbm.at[sl], r1, sem1)
            c2 = pltpu.async_copy(p2_hbm.at[sl], r2, sem2)
            cd = pltpu.async_copy(pd_hbm.at[didx.at[pl.ds(off, CH)]], rd,
                                  semd)
            c1.wait()
            pltpu.sync_copy(r1, g1_out.at[pl.ds(base + off, CH)])
            c2.wait()
            pltpu.sync_copy(r2, g2_out.at[pl.ds(base + off, CH)])
            cd.wait()
            pltpu.sync_copy(rd, gd_out.at[pl.ds(base + off, CH)])
            return carry

        lax.fori_loop(0, NCH, body, 0)

    return k(p_src1, p_src2, p_dst, src2, dst2)


def _pack(x, half):
    hi = x[:, :half].astype(jnp.bfloat16).astype(jnp.float32)
    lo = x[:, half:].astype(jnp.bfloat16).astype(jnp.float32)
    hi_w = lax.bitcast_convert_type(hi, jnp.int32)
    lo_w = lax.shift_right_logical(lax.bitcast_convert_type(lo, jnp.int32),
                                   16)
    return lax.bitwise_or(hi_w, lo_w)


def _unpack(w):
    hi = lax.bitcast_convert_type(
        lax.bitwise_and(w, jnp.int32(-65536)), jnp.float32)
    lo = lax.bitcast_convert_type(lax.shift_left(w, 16), jnp.float32)
    return hi, lo


def _proj_body(h_ref, wp1, bp1, wp2, wpd, p1_out, p2_out, pd_out):
    h = h_ref[...]
    m1 = h @ wp1[...] + bp1[...]
    p2 = h @ wp2[...]
    pd = h @ wpd[...]
    p1_out[...] = _pack(m1, PS1)
    p2_out[...] = _pack(p2, PS2)
    pd_out[...] = _pack(pd, PDP)


def _proj(h, wp1, bp1, wp2, wpd):
    return pl.pallas_call(
        _proj_body,
        grid=(NGRID,),
        in_specs=[pl.BlockSpec((NB, H), lambda i: (i, 0)),
                  _full(wp1), _full(bp1), _full(wp2), _full(wpd)],
        out_specs=(pl.BlockSpec((NB, PS1), lambda i: (i, 0)),
                   pl.BlockSpec((NB, PS2), lambda i: (i, 0)),
                   pl.BlockSpec((NB, PDP), lambda i: (i, 0))),
        out_shape=(jax.ShapeDtypeStruct((N, PS1), jnp.int32),
                   jax.ShapeDtypeStruct((N, PS2), jnp.int32),
                   jax.ShapeDtypeStruct((N, PDP), jnp.int32)),
    )(h, wp1, bp1, wp2, wpd)


def _sc_scatter(m, dst3, zeros_nh):
    mesh = plsc.VectorSubcoreMesh(core_axis_name="c", subcore_axis_name="s",
                                  num_cores=NC, num_subcores=NS)

    @functools.partial(
        pl.kernel,
        out_type=jax.ShapeDtypeStruct((2 * N, H), jnp.float32),
        mesh=mesh,
        scratch_types=[
            pltpu.VMEM((NCH, CH), jnp.int32),
            pltpu.VMEM((CH, H), jnp.float32),
            pltpu.VMEM_SHARED((N, H), jnp.float32),
        ],
    )
    def k(m_hbm, dst_hbm, zeros_hbm, out_hbm, idxs, rows, shared):
        cid = lax.axis_index("c")
        sid = lax.axis_index("s")
        wid = sid * NC + cid
        base = wid * EPW
        row0 = sid * RPT

        @pl.when(sid < NS - 1)
        def _():
            pltpu.sync_copy(zeros_hbm.at[pl.ds(row0, RPT)],
                            shared.at[pl.ds(row0, RPT)])

        @pl.when(sid == NS - 1)
        def _():
            pltpu.sync_copy(zeros_hbm.at[pl.ds((NS - 1) * RPT, RPT_LAST)],
                            shared.at[pl.ds((NS - 1) * RPT, RPT_LAST)])

        pltpu.sync_copy(dst_hbm.at[wid], idxs)
        plsc.subcore_barrier()

        def body(j, carry):
            pltpu.sync_copy(m_hbm.at[pl.ds(base + j * CH, CH)], rows)
            pltpu.sync_copy(rows, shared.at[idxs.at[j]], add=True)
            return carry

        lax.fori_loop(0, NCH, body, 0)
        plsc.subcore_barrier()

        obase = cid * N + row0

        @pl.when(sid < NS - 1)
        def _():
            pltpu.sync_copy(shared.at[pl.ds(row0, RPT)],
                            out_hbm.at[pl.ds(obase, RPT)])

        @pl.when(sid == NS - 1)
        def _():
            pltpu.sync_copy(shared.at[pl.ds((NS - 1) * RPT, RPT_LAST)],
                            out_hbm.at[pl.ds(cid * N + (NS - 1) * RPT,
                                             RPT_LAST)])

    return k(m, dst3, zeros_nh)


def _edge_layer_body(g1_ref, g2_ref, gd_ref, e_ref, wqe, bq, emw2, emb2,
                     eng, enb, w1e_lo, w1e_hi, w2_lo, w2_hi, b2,
                     e_out, m_out):
    m1_lo, m1_hi = _unpack(g1_ref[...])
    sa, sb = _unpack(g2_ref[...])
    da, db = _unpack(gd_ref[...])
    e = e_ref[...]
    qe = e @ wqe[...] + bq[...]
    glogit = sb[:, 0:1] + db[:, 0:1] + qe[:, EH:EH + 1]
    gate = jax.nn.sigmoid(glogit)
    dpre = sa + da + qe[:, :EH]
    delta = _gelu(dpre) @ emw2[...] + emb2[...]
    e2 = _ln(e + EDGE_SCALE * delta * gate, eng[...], enb[...])
    t_lo = _gelu(m1_lo + e2 @ w1e_lo[...])
    t_hi = _gelu(m1_hi + e2 @ w1e_hi[...])
    m_out[...] = t_lo @ w2_lo[...] + t_hi @ w2_hi[...] + b2[...]
    e_out[...] = e2


def _edge_layer(g1, g2, gd, e, weights):
    wspecs = [_full(w) for w in weights]
    return pl.pallas_call(
        _edge_layer_body,
        grid=(EGRID,),
        in_specs=[pl.BlockSpec((EB, PS1), lambda i: (i, 0)),
                  pl.BlockSpec((EB, PS2), lambda i: (i, 0)),
                  pl.BlockSpec((EB, PDP), lambda i: (i, 0)),
                  pl.BlockSpec((EB, EH), lambda i: (i, 0))] + wspecs,
        out_specs=(pl.BlockSpec((EB, EH), lambda i: (i, 0)),
                   pl.BlockSpec((EB, H), lambda i: (i, 0))),
        out_shape=(jax.ShapeDtypeStruct((E, EH), jnp.float32),
                   jax.ShapeDtypeStruct((E, H), jnp.float32)),
    )(g1, g2, gd, e, *weights)


def _node_update_body(h_ref, a0_ref, a1_ref, w1h, w1a, b1, w2, b2, ng, nb,
                      out_ref):
    h = h_ref[...]
    agg = a0_ref[...] + a1_ref[...]
    u = _gelu(h @ w1h[...] + agg @ w1a[...] + b1[...]) @ w2[...] + b2[...]
    out_ref[...] = _ln(h + u, ng[...], nb[...])


def _node_update(h, a0, a1, weights):
    wspecs = [_full(w) for w in weights]
    return pl.pallas_call(
        _node_update_body,
        grid=(NGRID,),
        in_specs=[pl.BlockSpec((NB, H), lambda i: (i, 0)),
                  pl.BlockSpec((NB, H), lambda i: (i, 0)),
                  pl.BlockSpec((NB, H), lambda i: (i, 0))] + wspecs,
        out_specs=pl.BlockSpec((NB, H), lambda i: (i, 0)),
        out_shape=jax.ShapeDtypeStruct((N, H), jnp.float32),
    )(h, a0, a1, *weights)


def _final_body(h_ref, b_ref, og, ob, h_out, mix_out, summ, cnt):
    i = pl.program_id(0)
    hn = _ln(h_ref[...], og[...], ob[...])
    h_out[...] = hn

    @pl.when(i == 0)
    def _():
        summ[...] = jnp.zeros_like(summ)
        cnt[...] = jnp.zeros_like(cnt)

    bids = b_ref[0, 0, :]
    gid = lax.broadcasted_iota(jnp.int32, (G, FB), 0)
    S = (gid == bids[None, :]).astype(jnp.float32)
    summ[...] += jnp.dot(S, hn)
    cnt[...] += jnp.dot(S, jnp.ones((FB, H), jnp.float32))

    @pl.when(i == FGRID - 1)
    def _():
        mix_out[...] = summ[...] / jnp.clip(cnt[...], 1.0, None)


def _final_pool(h, batch3, og, ob):
    return pl.pallas_call(
        _final_body,
        grid=(FGRID,),
        in_specs=[pl.BlockSpec((FB, H), lambda i: (i, 0)),
                  pl.BlockSpec((1, 1, FB), lambda i: (i, 0, 0)),
                  _full(og), _full(ob)],
        out_specs=(pl.BlockSpec((FB, H), lambda i: (i, 0)),
                   pl.BlockSpec((G, H), lambda i: (0, 0))),
        out_shape=(jax.ShapeDtypeStruct((N, H), jnp.float32),
                   jax.ShapeDtypeStruct((G, H), jnp.float32)),
        scratch_shapes=[pltpu.VMEM((G, H), jnp.float32),
                        pltpu.VMEM((G, H), jnp.float32)],
    )(h, batch3, og, ob)


def _row(x):
    return x.reshape(1, -1).astype(jnp.float32)


def kernel(node_h, edge_index, edge_attr, batch, fallback_num_graphs, params):
    src = edge_index[0].astype(jnp.int32)
    dst = edge_index[1].astype(jnp.int32)
    src2 = src.reshape(NW, EPW)
    dst2 = dst.reshape(NW, EPW)
    dst3 = dst.reshape(NW, NCH, CH)
    batch3 = batch.astype(jnp.int32).reshape(FGRID, 1, FB)
    zeros_nh = jnp.zeros((N, H), jnp.float32)

    e = _edge_in(edge_attr, params['edge_in_W'], _row(params['edge_in_b']),
                 _row(params['edge_norm_g']), _row(params['edge_norm_b']))

    h = node_h
    for lp in params['layers']:
        em_W1 = lp['em_W1']
        eg_W = lp['eg_W']
        wp1 = lp['msg_W1'][:H]
        bp1 = _row(lp['msg_b1'])
        wp2 = jnp.zeros((H, 2 * PS2), jnp.float32)
        wp2 = wp2.at[:, :EH].set(em_W1[:H])
        wp2 = wp2.at[:, EH].set(eg_W[:H, 0])
        wpd = jnp.zeros((H, 2 * PDP), jnp.float32)
        wpd = wpd.at[:, :EH].set(em_W1[H:2 * H])
        wpd = wpd.at[:, EH].set(eg_W[H:2 * H, 0])
        wqe = jnp.zeros((EH, 32), jnp.float32)
        wqe = wqe.at[:, :EH].set(em_W1[2 * H:]).at[:, EH].set(eg_W[2 * H:, 0])
        bq = jnp.zeros((1, 32), jnp.float32)
        bq = bq.at[0, :EH].set(lp['em_b1']).at[0, EH].set(lp['eg_b'][0])

        w1e = lp['msg_W1'][H:]
        edge_w = [wqe, bq, lp['em_W2'], _row(lp['em_b2']),
                  _row(lp['enorm_g']), _row(lp['enorm_b']),
                  w1e[:, :64], w1e[:, 64:],
                  lp['msg_W2'][:64], lp['msg_W2'][64:], _row(lp['msg_b2'])]
        upd_w = [lp['upd_W1'][:H], lp['upd_W1'][H:], _row(lp['upd_b1']),
                 lp['upd_W2'], _row(lp['upd_b2']),
                 _row(lp['norm_g']), _row(lp['norm_b'])]

        p1, p2, pd = _proj(h, wp1, bp1, wp2, wpd)
        g1, g2, gd = _sc_gather(p1, p2, pd, src2, dst2)
        e, m = _edge_layer(g1, g2, gd, e, edge_w)
        parts = _sc_scatter(m, dst3, zeros_nh)
        h = _node_update(h, parts[:N], parts[N:], upd_w)

    h_out, mix = _final_pool(h, batch3, _row(params['out_norm_g']),
                             _row(params['out_norm_b']))
    scale = fallback_num_graphs.astype(jnp.float32) / jnp.float32(G) \
        if hasattr(fallback_num_graphs, 'astype') \
        else jnp.float32(fallback_num_graphs) / jnp.float32(G)
    mix = mix * scale
    return h_out, mix


# speedup vs baseline: 1.0615x; 1.0615x over previous
import functools

import jax
import jax.numpy as jnp
from jax import lax
from jax.experimental import pallas as pl
from jax.experimental.pallas import tpu as pltpu
from jax.experimental.pallas import tpu_sc as plsc

H = 128
EH = 16
E = 320000
N = 10000
G = 2500
EDGE_SCALE = 0.1

NC = 2
NS = 16
NW = NC * NS
EPW = E // NW
CH = 80
NCH = EPW // CH
RPT = 632
RPT_LAST = N - (NS - 1) * RPT

EB = 2560
EGRID = E // EB
NB = 2000
NGRID = N // NB
FB = 400
FGRID = N // FB

PS1 = 64
PS2 = 16
PDP = 16


def _gelu(x):
    return 0.5 * x * (1.0 + lax.erf(x * 0.7071067811865476))


def _ln(x, g, b, eps=1e-5):
    mu = jnp.mean(x, axis=-1, keepdims=True)
    var = jnp.var(x, axis=-1, keepdims=True)
    return (x - mu) / jnp.sqrt(var + eps) * g + b


_RED16 = None


def _ln16(x, g, b, eps=1e-5):
    red = jnp.full((EH, EH), 1.0 / EH, jnp.float32)
    mu = x @ red
    var = (x * x) @ red - mu * mu
    return (x - mu) * lax.rsqrt(var + eps) * g + b


def _full(x):
    return pl.BlockSpec(x.shape, lambda *_: (0,) * x.ndim)


def _sc_gather(p_src1, p_src2, p_dst, src2, dst2):
    mesh = plsc.VectorSubcoreMesh(core_axis_name="c", subcore_axis_name="s",
                                  num_cores=NC, num_subcores=NS)

    @functools.partial(
        pl.kernel,
        out_type=(jax.ShapeDtypeStruct((E, PS1), jnp.int32),
                  jax.ShapeDtypeStruct((E, PS2), jnp.int32),
                  jax.ShapeDtypeStruct((E, PDP), jnp.int32)),
        mesh=mesh,
        scratch_types=[
            pltpu.VMEM((EPW,), jnp.int32),
            pltpu.VMEM((EPW,), jnp.int32),
            pltpu.VMEM((CH, PS1), jnp.int32),
            pltpu.VMEM((CH, PS2), jnp.int32),
            pltpu.VMEM((CH, PDP), jnp.int32),
            pltpu.SemaphoreType.DMA,
            pltpu.SemaphoreType.DMA,
            pltpu.SemaphoreType.DMA,
        ],
        compiler_params=pltpu.CompilerParams(use_tc_tiling_on_sc=False),
    )
    def k(p1_hbm, p2_hbm, pd_hbm, src_hbm, dst_hbm, g1_out, g2_out, gd_out,
          sidx, didx, r1, r2, rd, sem1, sem2, semd):
        wid = lax.axis_index("s") * NC + lax.axis_index("c")
        base = wid * EPW
        pltpu.sync_copy(src_hbm.at[wid], sidx)
        pltpu.sync_copy(dst_hbm.at[wid], didx)

        def body(j, carry):
            off = j * CH
            sl = sidx.at[pl.ds(off, CH)]
            c1 = pltpu.async_copy(p1_hbm.at[sl], r1, sem1)
            c2 = pltpu.async_copy(p2_hbm.at[sl], r2, sem2)
            cd = pltpu.async_copy(pd_hbm.at[didx.at[pl.ds(off, CH)]], rd,
                                  semd)
            c1.wait()
            pltpu.sync_copy(r1, g1_out.at[pl.ds(base + off, CH)])
            c2.wait()
            pltpu.sync_copy(r2, g2_out.at[pl.ds(base + off, CH)])
            cd.wait()
            pltpu.sync_copy(rd, gd_out.at[pl.ds(base + off, CH)])
            return carry

        lax.fori_loop(0, NCH, body, 0)

    return k(p_src1, p_src2, p_dst, src2, dst2)


def _pack(x, half):
    hi = x[:, :half].astype(jnp.bfloat16).astype(jnp.float32)
    lo = x[:, half:].astype(jnp.bfloat16).astype(jnp.float32)
    hi_w = lax.bitcast_convert_type(hi, jnp.int32)
    lo_w = lax.shift_right_logical(lax.bitcast_convert_type(lo, jnp.int32),
                                   16)
    return lax.bitwise_or(hi_w, lo_w)


def _unpack(w):
    hi = lax.bitcast_convert_type(
        lax.bitwise_and(w, jnp.int32(-65536)), jnp.float32)
    lo = lax.bitcast_convert_type(lax.shift_left(w, 16), jnp.float32)
    return hi, lo


def _proj_body(h_ref, wp1, bp1, wp2, wpd, p1_out, p2_out, pd_out):
    h = h_ref[...]
    m1 = h @ wp1[...] + bp1[...]
    p2 = h @ wp2[...]
    pd = h @ wpd[...]
    p1_out[...] = _pack(m1, PS1)
    p2_out[...] = _pack(p2, PS2)
    pd_out[...] = _pack(pd, PDP)


def _proj(h, wp1, bp1, wp2, wpd):
    return pl.pallas_call(
        _proj_body,
        grid=(NGRID,),
        in_specs=[pl.BlockSpec((NB, H), lambda i: (i, 0)),
                  _full(wp1), _full(bp1), _full(wp2), _full(wpd)],
        out_specs=(pl.BlockSpec((NB, PS1), lambda i: (i, 0)),
                   pl.BlockSpec((NB, PS2), lambda i: (i, 0)),
                   pl.BlockSpec((NB, PDP), lambda i: (i, 0))),
        out_shape=(jax.ShapeDtypeStruct((N, PS1), jnp.int32),
                   jax.ShapeDtypeStruct((N, PS2), jnp.int32),
                   jax.ShapeDtypeStruct((N, PDP), jnp.int32)),
    )(h, wp1, bp1, wp2, wpd)


def _sc_scatter(m, dst3, zeros_nh):
    mesh = plsc.VectorSubcoreMesh(core_axis_name="c", subcore_axis_name="s",
                                  num_cores=NC, num_subcores=NS)

    @functools.partial(
        pl.kernel,
        out_type=jax.ShapeDtypeStruct((2 * N, H), jnp.float32),
        mesh=mesh,
        scratch_types=[
            pltpu.VMEM((NCH, CH), jnp.int32),
            pltpu.VMEM((CH, H), jnp.float32),
            pltpu.VMEM_SHARED((N, H), jnp.float32),
        ],
    )
    def k(m_hbm, dst_hbm, zeros_hbm, out_hbm, idxs, rows, shared):
        cid = lax.axis_index("c")
        sid = lax.axis_index("s")
        wid = sid * NC + cid
        base = wid * EPW
        row0 = sid * RPT

        @pl.when(sid < NS - 1)
        def _():
            pltpu.sync_copy(zeros_hbm.at[pl.ds(row0, RPT)],
                            shared.at[pl.ds(row0, RPT)])

        @pl.when(sid == NS - 1)
        def _():
            pltpu.sync_copy(zeros_hbm.at[pl.ds((NS - 1) * RPT, RPT_LAST)],
                            shared.at[pl.ds((NS - 1) * RPT, RPT_LAST)])

        pltpu.sync_copy(dst_hbm.at[wid], idxs)
        plsc.subcore_barrier()

        def body(j, carry):
            pltpu.sync_copy(m_hbm.at[pl.ds(base + j * CH, CH)], rows)
            pltpu.sync_copy(rows, shared.at[idxs.at[j]], add=True)
            return carry

        lax.fori_loop(0, NCH, body, 0)
        plsc.subcore_barrier()

        obase = cid * N + row0

        @pl.when(sid < NS - 1)
        def _():
            pltpu.sync_copy(shared.at[pl.ds(row0, RPT)],
                            out_hbm.at[pl.ds(obase, RPT)])

        @pl.when(sid == NS - 1)
        def _():
            pltpu.sync_copy(shared.at[pl.ds((NS - 1) * RPT, RPT_LAST)],
                            out_hbm.at[pl.ds(cid * N + (NS - 1) * RPT,
                                             RPT_LAST)])

    return k(m, dst3, zeros_nh)


def _make_edge_body(first):
    def body(g1_ref, g2_ref, gd_ref, e_ref, wqe, bq, emw2, emb2,
             eng, enb, w1e, w2, b2, ein_w, ein_b, ein_g, ein_bb,
             e_out, m_out):
        m1_lo, m1_hi = _unpack(g1_ref[...])
        sa, sb = _unpack(g2_ref[...])
        da, db = _unpack(gd_ref[...])
        if first:
            x = e_ref[...] @ ein_w[...] + ein_b[...]
            e = _ln16(_gelu(x), ein_g[...], ein_bb[...])
        else:
            e = e_ref[...]
        qe = e @ wqe[...] + bq[...]
        glogit = sb[:, 0:1] + db[:, 0:1] + qe[:, EH:EH + 1]
        gate = jax.nn.sigmoid(glogit)
        dpre = sa + da + qe[:, :EH]
        delta = _gelu(dpre) @ emw2[...] + emb2[...]
        e2 = _ln16(e + EDGE_SCALE * delta * gate, eng[...], enb[...])
        m1 = jnp.concatenate([m1_lo, m1_hi], axis=1)
        t = _gelu(m1 + e2 @ w1e[...])
        m_out[...] = t @ w2[...] + b2[...]
        e_out[...] = e2
    return body


def _edge_layer(g1, g2, gd, e, weights, first):
    wspecs = [_full(w) for w in weights]
    return pl.pallas_call(
        _make_edge_body(first),
        grid=(EGRID,),
        in_specs=[pl.BlockSpec((EB, PS1), lambda i: (i, 0)),
                  pl.BlockSpec((EB, PS2), lambda i: (i, 0)),
                  pl.BlockSpec((EB, PDP), lambda i: (i, 0)),
                  pl.BlockSpec((EB, EH), lambda i: (i, 0))] + wspecs,
        out_specs=(pl.BlockSpec((EB, EH), lambda i: (i, 0)),
                   pl.BlockSpec((EB, H), lambda i: (i, 0))),
        out_shape=(jax.ShapeDtypeStruct((E, EH), jnp.float32),
                   jax.ShapeDtypeStruct((E, H), jnp.float32)),
    )(g1, g2, gd, e, *weights)


def _node_update_body(h_ref, a0_ref, a1_ref, w1h, w1a, b1, w2, b2, ng, nb,
                      out_ref):
    h = h_ref[...]
    agg = a0_ref[...] + a1_ref[...]
    u = _gelu(h @ w1h[...] + agg @ w1a[...] + b1[...]) @ w2[...] + b2[...]
    out_ref[...] = _ln(h + u, ng[...], nb[...])


def _node_update(h, a0, a1, weights):
    wspecs = [_full(w) for w in weights]
    return pl.pallas_call(
        _node_update_body,
        grid=(NGRID,),
        in_specs=[pl.BlockSpec((NB, H), lambda i: (i, 0)),
                  pl.BlockSpec((NB, H), lambda i: (i, 0)),
                  pl.BlockSpec((NB, H), lambda i: (i, 0))] + wspecs,
        out_specs=pl.BlockSpec((NB, H), lambda i: (i, 0)),
        out_shape=jax.ShapeDtypeStruct((N, H), jnp.float32),
    )(h, a0, a1, *weights)


def _final_body(h_ref, b_ref, og, ob, h_out, mix_out, summ, cnt):
    i = pl.program_id(0)
    hn = _ln(h_ref[...], og[...], ob[...])
    h_out[...] = hn

    @pl.when(i == 0)
    def _():
        summ[...] = jnp.zeros_like(summ)
        cnt[...] = jnp.zeros_like(cnt)

    bids = b_ref[0, 0, :]
    gid = lax.broadcasted_iota(jnp.int32, (G, FB), 0)
    S = (gid == bids[None, :]).astype(jnp.float32)
    summ[...] += jnp.dot(S, hn)
    cnt[...] += jnp.dot(S, jnp.ones((FB, H), jnp.float32))

    @pl.when(i == FGRID - 1)
    def _():
        mix_out[...] = summ[...] / jnp.clip(cnt[...], 1.0, None)


def _final_pool(h, batch3, og, ob):
    return pl.pallas_call(
        _final_body,
        grid=(FGRID,),
        in_specs=[pl.BlockSpec((FB, H), lambda i: (i, 0)),
                  pl.BlockSpec((1, 1, FB), lambda i: (i, 0, 0)),
                  _full(og), _full(ob)],
        out_specs=(pl.BlockSpec((FB, H), lambda i: (i, 0)),
                   pl.BlockSpec((G, H), lambda i: (0, 0))),
        out_shape=(jax.ShapeDtypeStruct((N, H), jnp.float32),
                   jax.ShapeDtypeStruct((G, H), jnp.float32)),
        scratch_shapes=[pltpu.VMEM((G, H), jnp.float32),
                        pltpu.VMEM((G, H), jnp.float32)],
    )(h, batch3, og, ob)


def _row(x):
    return x.reshape(1, -1).astype(jnp.float32)


def kernel(node_h, edge_index, edge_attr, batch, fallback_num_graphs, params):
    src = edge_index[0].astype(jnp.int32)
    dst = edge_index[1].astype(jnp.int32)
    src2 = src.reshape(NW, EPW)
    dst2 = dst.reshape(NW, EPW)
    dst3 = dst.reshape(NW, NCH, CH)
    batch3 = batch.astype(jnp.int32).reshape(FGRID, 1, FB)
    zeros_nh = jnp.zeros((N, H), jnp.float32)

    e = edge_attr
    ein = [params['edge_in_W'], _row(params['edge_in_b']),
           _row(params['edge_norm_g']), _row(params['edge_norm_b'])]
    h = node_h
    for li, lp in enumerate(params['layers']):
        em_W1 = lp['em_W1']
        eg_W = lp['eg_W']
        wp1 = lp['msg_W1'][:H]
        bp1 = _row(lp['msg_b1'])
        wp2 = jnp.zeros((H, 2 * PS2), jnp.float32)
        wp2 = wp2.at[:, :EH].set(em_W1[:H])
        wp2 = wp2.at[:, EH].set(eg_W[:H, 0])
        wpd = jnp.zeros((H, 2 * PDP), jnp.float32)
        wpd = wpd.at[:, :EH].set(em_W1[H:2 * H])
        wpd = wpd.at[:, EH].set(eg_W[H:2 * H, 0])
        wqe = jnp.zeros((EH, 32), jnp.float32)
        wqe = wqe.at[:, :EH].set(em_W1[2 * H:]).at[:, EH].set(eg_W[2 * H:, 0])
        bq = jnp.zeros((1, 32), jnp.float32)
        bq = bq.at[0, :EH].set(lp['em_b1']).at[0, EH].set(lp['eg_b'][0])

        edge_w = [wqe, bq, lp['em_W2'], _row(lp['em_b2']),
                  _row(lp['enorm_g']), _row(lp['enorm_b']),
                  lp['msg_W1'][H:], lp['msg_W2'], _row(lp['msg_b2'])] + ein
        upd_w = [lp['upd_W1'][:H], lp['upd_W1'][H:], _row(lp['upd_b1']),
                 lp['upd_W2'], _row(lp['upd_b2']),
                 _row(lp['norm_g']), _row(lp['norm_b'])]

        p1, p2, pd = _proj(h, wp1, bp1, wp2, wpd)
        g1, g2, gd = _sc_gather(p1, p2, pd, src2, dst2)
        e, m = _edge_layer(g1, g2, gd, e, edge_w, li == 0)
        parts = _sc_scatter(m, dst3, zeros_nh)
        h = _node_update(h, parts[:N], parts[N:], upd_w)

    h_out, mix = _final_pool(h, batch3, _row(params['out_norm_g']),
                             _row(params['out_norm_b']))
    scale = fallback_num_graphs.astype(jnp.float32) / jnp.float32(G) \
        if hasattr(fallback_num_graphs, 'astype') \
        else jnp.float32(fallback_num_graphs) / jnp.float32(G)
    mix = mix * scale
    return h_out, mix


# speedup vs baseline: 1.4371x; 1.3539x over previous
import functools

import jax
import jax.numpy as jnp
from jax import lax
from jax.experimental import pallas as pl
from jax.experimental.pallas import tpu as pltpu
from jax.experimental.pallas import tpu_sc as plsc

H = 128
EH = 16
E = 320000
N = 10000
G = 2500
EDGE_SCALE = 0.1

NC = 2
NS = 16
NW = NC * NS
EPW = E // NW
CH = 80
NCH = EPW // CH
RPT = 632
RPT_LAST = N - (NS - 1) * RPT

EB = 2560
EGRID = E // EB
NB = 2000
NGRID = N // NB
FB = 400
FGRID = N // FB

PS1 = 64
PS2 = 16
PDP = 16


def _gelu(x):
    return 0.5 * x * (1.0 + lax.erf(x * 0.7071067811865476))


def _ln(x, g, b, eps=1e-5):
    mu = jnp.mean(x, axis=-1, keepdims=True)
    var = jnp.var(x, axis=-1, keepdims=True)
    return (x - mu) / jnp.sqrt(var + eps) * g + b


_RED16 = None


def _ln16(x, g, b, eps=1e-5):
    red = jnp.full((EH, EH), 1.0 / EH, jnp.float32)
    mu = x @ red
    var = (x * x) @ red - mu * mu
    return (x - mu) * lax.rsqrt(var + eps) * g + b


def _full(x):
    return pl.BlockSpec(x.shape, lambda *_: (0,) * x.ndim)


def _sc_gather(p_src1, p_src2, p_dst, src2, dst2):
    mesh = plsc.VectorSubcoreMesh(core_axis_name="c", subcore_axis_name="s",
                                  num_cores=NC, num_subcores=NS)

    @functools.partial(
        pl.kernel,
        out_type=jax.ShapeDtypeStruct((E, H), jnp.int32),
        mesh=mesh,
        scratch_types=[
            pltpu.VMEM((EPW,), jnp.int32),
            pltpu.VMEM((EPW,), jnp.int32),
            pltpu.VMEM((CH, PS1), jnp.int32),
            pltpu.VMEM((CH, PS2), jnp.int32),
            pltpu.VMEM((CH, PDP), jnp.int32),
            pltpu.SemaphoreType.DMA,
            pltpu.SemaphoreType.DMA,
            pltpu.SemaphoreType.DMA,
        ],
        compiler_params=pltpu.CompilerParams(use_tc_tiling_on_sc=False),
    )
    def k(p1_hbm, p2_hbm, pd_hbm, src_hbm, dst_hbm, g_out,
          sidx, didx, r1, r2, rd, sem1, sem2, semd):
        wid = lax.axis_index("s") * NC + lax.axis_index("c")
        base = wid * EPW
        pltpu.sync_copy(src_hbm.at[wid], sidx)
        pltpu.sync_copy(dst_hbm.at[wid], didx)

        def body(j, carry):
            off = j * CH
            sl = sidx.at[pl.ds(off, CH)]
            c1 = pltpu.async_copy(p1_hbm.at[sl], r1, sem1)
            c2 = pltpu.async_copy(p2_hbm.at[sl], r2, sem2)
            cd = pltpu.async_copy(pd_hbm.at[didx.at[pl.ds(off, CH)]], rd,
                                  semd)
            rows = g_out.at[pl.ds(base + off, CH)]
            c1.wait()
            pltpu.sync_copy(r1, rows.at[:, pl.ds(0, PS1)])
            c2.wait()
            pltpu.sync_copy(r2, rows.at[:, pl.ds(PS1, PS2)])
            cd.wait()
            pltpu.sync_copy(rd, rows.at[:, pl.ds(PS1 + PS2, PDP)])
            return carry

        lax.fori_loop(0, NCH, body, 0)

    return k(p_src1, p_src2, p_dst, src2, dst2)


def _pack(x, half):
    hi = x[:, :half].astype(jnp.bfloat16).astype(jnp.float32)
    lo = x[:, half:].astype(jnp.bfloat16).astype(jnp.float32)
    hi_w = lax.bitcast_convert_type(hi, jnp.int32)
    lo_w = lax.shift_right_logical(lax.bitcast_convert_type(lo, jnp.int32),
                                   16)
    return lax.bitwise_or(hi_w, lo_w)


def _unpack(w):
    hi = lax.bitcast_convert_type(
        lax.bitwise_and(w, jnp.int32(-65536)), jnp.float32)
    lo = lax.bitcast_convert_type(lax.shift_left(w, 16), jnp.float32)
    return hi, lo


def _proj_body(h_ref, wp1, bp1, wp2, wpd, p1_out, p2_out, pd_out):
    h = h_ref[...]
    m1 = h @ wp1[...] + bp1[...]
    p2 = h @ wp2[...]
    pd = h @ wpd[...]
    p1_out[...] = _pack(m1, PS1)
    p2_out[...] = _pack(p2, PS2)
    pd_out[...] = _pack(pd, PDP)


def _proj(h, wp1, bp1, wp2, wpd):
    return pl.pallas_call(
        _proj_body,
        grid=(NGRID,),
        in_specs=[pl.BlockSpec((NB, H), lambda i: (i, 0)),
                  _full(wp1), _full(bp1), _full(wp2), _full(wpd)],
        out_specs=(pl.BlockSpec((NB, PS1), lambda i: (i, 0)),
                   pl.BlockSpec((NB, PS2), lambda i: (i, 0)),
                   pl.BlockSpec((NB, PDP), lambda i: (i, 0))),
        out_shape=(jax.ShapeDtypeStruct((N, PS1), jnp.int32),
                   jax.ShapeDtypeStruct((N, PS2), jnp.int32),
                   jax.ShapeDtypeStruct((N, PDP), jnp.int32)),
    )(h, wp1, bp1, wp2, wpd)


def _sc_scatter(m, dst3, zeros_nh):
    mesh = plsc.VectorSubcoreMesh(core_axis_name="c", subcore_axis_name="s",
                                  num_cores=NC, num_subcores=NS)

    @functools.partial(
        pl.kernel,
        out_type=jax.ShapeDtypeStruct((2 * N, H), jnp.float32),
        mesh=mesh,
        scratch_types=[
            pltpu.VMEM((NCH, CH), jnp.int32),
            pltpu.VMEM((CH, H), jnp.float32),
            pltpu.VMEM_SHARED((N, H), jnp.float32),
        ],
    )
    def k(m_hbm, dst_hbm, zeros_hbm, out_hbm, idxs, rows, shared):
        cid = lax.axis_index("c")
        sid = lax.axis_index("s")
        wid = sid * NC + cid
        base = wid * EPW
        row0 = sid * RPT

        @pl.when(sid < NS - 1)
        def _():
            pltpu.sync_copy(zeros_hbm.at[pl.ds(row0, RPT)],
                            shared.at[pl.ds(row0, RPT)])

        @pl.when(sid == NS - 1)
        def _():
            pltpu.sync_copy(zeros_hbm.at[pl.ds((NS - 1) * RPT, RPT_LAST)],
                            shared.at[pl.ds((NS - 1) * RPT, RPT_LAST)])

        pltpu.sync_copy(dst_hbm.at[wid], idxs)
        plsc.subcore_barrier()

        def body(j, carry):
            pltpu.sync_copy(m_hbm.at[pl.ds(base + j * CH, CH)], rows)
            pltpu.sync_copy(rows, shared.at[idxs.at[j]], add=True)
            return carry

        lax.fori_loop(0, NCH, body, 0)
        plsc.subcore_barrier()

        obase = cid * N + row0

        @pl.when(sid < NS - 1)
        def _():
            pltpu.sync_copy(shared.at[pl.ds(row0, RPT)],
                            out_hbm.at[pl.ds(obase, RPT)])

        @pl.when(sid == NS - 1)
        def _():
            pltpu.sync_copy(shared.at[pl.ds((NS - 1) * RPT, RPT_LAST)],
                            out_hbm.at[pl.ds(cid * N + (NS - 1) * RPT,
                                             RPT_LAST)])

    return k(m, dst3, zeros_nh)


def _make_edge_body(first):
    def body(g_ref, e_ref, wqe, bq, emw2, emb2,
             eng, enb, w1e, w2, b2, ein_w, ein_b, ein_g, ein_bb,
             e_out, m_out):
        gall = g_ref[...]
        m1_lo, m1_hi = _unpack(gall[:, :PS1])
        sa, sb = _unpack(gall[:, PS1:PS1 + PS2])
        da, db = _unpack(gall[:, PS1 + PS2:PS1 + PS2 + PDP])
        if first:
            x = e_ref[...] @ ein_w[...] + ein_b[...]
            e = _ln16(_gelu(x), ein_g[...], ein_bb[...])
        else:
            e = e_ref[...]
        qe = e @ wqe[...] + bq[...]
        glogit = sb[:, 0:1] + db[:, 0:1] + qe[:, EH:EH + 1]
        gate = jax.nn.sigmoid(glogit)
        dpre = sa + da + qe[:, :EH]
        delta = _gelu(dpre) @ emw2[...] + emb2[...]
        e2 = _ln16(e + EDGE_SCALE * delta * gate, eng[...], enb[...])
        m1 = jnp.concatenate([m1_lo, m1_hi], axis=1)
        t = _gelu(m1 + e2 @ w1e[...])
        m_out[...] = t @ w2[...] + b2[...]
        e_out[...] = e2
    return body


def _edge_layer(gall, e, weights, first):
    wspecs = [_full(w) for w in weights]
    return pl.pallas_call(
        _make_edge_body(first),
        grid=(EGRID,),
        in_specs=[pl.BlockSpec((EB, H), lambda i: (i, 0)),
                  pl.BlockSpec((EB, EH), lambda i: (i, 0))] + wspecs,
        out_specs=(pl.BlockSpec((EB, EH), lambda i: (i, 0)),
                   pl.BlockSpec((EB, H), lambda i: (i, 0))),
        out_shape=(jax.ShapeDtypeStruct((E, EH), jnp.float32),
                   jax.ShapeDtypeStruct((E, H), jnp.float32)),
    )(gall, e, *weights)


def _node_update_body(h_ref, a0_ref, a1_ref, w1h, w1a, b1, w2, b2, ng, nb,
                      out_ref):
    h = h_ref[...]
    agg = a0_ref[...] + a1_ref[...]
    u = _gelu(h @ w1h[...] + agg @ w1a[...] + b1[...]) @ w2[...] + b2[...]
    out_ref[...] = _ln(h + u, ng[...], nb[...])


def _node_update(h, a0, a1, weights):
    wspecs = [_full(w) for w in weights]
    return pl.pallas_call(
        _node_update_body,
        grid=(NGRID,),
        in_specs=[pl.BlockSpec((NB, H), lambda i: (i, 0)),
                  pl.BlockSpec((NB, H), lambda i: (i, 0)),
                  pl.BlockSpec((NB, H), lambda i: (i, 0))] + wspecs,
        out_specs=pl.BlockSpec((NB, H), lambda i: (i, 0)),
        out_shape=jax.ShapeDtypeStruct((N, H), jnp.float32),
    )(h, a0, a1, *weights)


def _final_body(h_ref, b_ref, og, ob, h_out, mix_out, summ, cnt):
    i = pl.program_id(0)
    hn = _ln(h_ref[...], og[...], ob[...])
    h_out[...] = hn

    @pl.when(i == 0)
    def _():
        summ[...] = jnp.zeros_like(summ)
        cnt[...] = jnp.zeros_like(cnt)

    bids = b_ref[0, 0, :]
    gid = lax.broadcasted_iota(jnp.int32, (G, FB), 0)
    S = (gid == bids[None, :]).astype(jnp.float32)
    summ[...] += jnp.dot(S, hn)
    cnt[...] += jnp.dot(S, jnp.ones((FB, H), jnp.float32))

    @pl.when(i == FGRID - 1)
    def _():
        mix_out[...] = summ[...] / jnp.clip(cnt[...], 1.0, None)


def _final_pool(h, batch3, og, ob):
    return pl.pallas_call(
        _final_body,
        grid=(FGRID,),
        in_specs=[pl.BlockSpec((FB, H), lambda i: (i, 0)),
                  pl.BlockSpec((1, 1, FB), lambda i: (i, 0, 0)),
                  _full(og), _full(ob)],
        out_specs=(pl.BlockSpec((FB, H), lambda i: (i, 0)),
                   pl.BlockSpec((G, H), lambda i: (0, 0))),
        out_shape=(jax.ShapeDtypeStruct((N, H), jnp.float32),
                   jax.ShapeDtypeStruct((G, H), jnp.float32)),
        scratch_shapes=[pltpu.VMEM((G, H), jnp.float32),
                        pltpu.VMEM((G, H), jnp.float32)],
    )(h, batch3, og, ob)


def _row(x):
    return x.reshape(1, -1).astype(jnp.float32)


def kernel(node_h, edge_index, edge_attr, batch, fallback_num_graphs, params):
    src = edge_index[0].astype(jnp.int32)
    dst = edge_index[1].astype(jnp.int32)
    src2 = src.reshape(NW, EPW)
    dst2 = dst.reshape(NW, EPW)
    dst3 = dst.reshape(NW, NCH, CH)
    batch3 = batch.astype(jnp.int32).reshape(FGRID, 1, FB)
    zeros_nh = jnp.zeros((N, H), jnp.float32)

    e = edge_attr
    ein = [params['edge_in_W'], _row(params['edge_in_b']),
           _row(params['edge_norm_g']), _row(params['edge_norm_b'])]
    h = node_h
    for li, lp in enumerate(params['layers']):
        em_W1 = lp['em_W1']
        eg_W = lp['eg_W']
        wp1 = lp['msg_W1'][:H]
        bp1 = _row(lp['msg_b1'])
        wp2 = jnp.zeros((H, 2 * PS2), jnp.float32)
        wp2 = wp2.at[:, :EH].set(em_W1[:H])
        wp2 = wp2.at[:, EH].set(eg_W[:H, 0])
        wpd = jnp.zeros((H, 2 * PDP), jnp.float32)
        wpd = wpd.at[:, :EH].set(em_W1[H:2 * H])
        wpd = wpd.at[:, EH].set(eg_W[H:2 * H, 0])
        wqe = jnp.zeros((EH, 32), jnp.float32)
        wqe = wqe.at[:, :EH].set(em_W1[2 * H:]).at[:, EH].set(eg_W[2 * H:, 0])
        bq = jnp.zeros((1, 32), jnp.float32)
        bq = bq.at[0, :EH].set(lp['em_b1']).at[0, EH].set(lp['eg_b'][0])

        edge_w = [wqe, bq, lp['em_W2'], _row(lp['em_b2']),
                  _row(lp['enorm_g']), _row(lp['enorm_b']),
                  lp['msg_W1'][H:], lp['msg_W2'], _row(lp['msg_b2'])] + ein
        upd_w = [lp['upd_W1'][:H], lp['upd_W1'][H:], _row(lp['upd_b1']),
                 lp['upd_W2'], _row(lp['upd_b2']),
                 _row(lp['norm_g']), _row(lp['norm_b'])]

        p1, p2, pd = _proj(h, wp1, bp1, wp2, wpd)
        gall = _sc_gather(p1, p2, pd, src2, dst2)
        e, m = _edge_layer(gall, e, edge_w, li == 0)
        parts = _sc_scatter(m, dst3, zeros_nh)
        h = _node_update(h, parts[:N], parts[N:], upd_w)

    h_out, mix = _final_pool(h, batch3, _row(params['out_norm_g']),
                             _row(params['out_norm_b']))
    scale = fallback_num_graphs.astype(jnp.float32) / jnp.float32(G) \
        if hasattr(fallback_num_graphs, 'astype') \
        else jnp.float32(fallback_num_graphs) / jnp.float32(G)
    mix = mix * scale
    return h_out, mix


# speedup vs baseline: 1.6478x; 1.1466x over previous
import functools

import jax
import jax.numpy as jnp
from jax import lax
from jax.experimental import pallas as pl
from jax.experimental.pallas import tpu as pltpu
from jax.experimental.pallas import tpu_sc as plsc

H = 128
EH = 16
E = 320000
N = 10000
G = 2500
EDGE_SCALE = 0.1

NC = 2
NS = 16
NW = NC * NS
EPW = E // NW
CH = 80
NCH = EPW // CH
GNB = 5
RPT = 632
RPT_LAST = N - (NS - 1) * RPT

EB = 2560
EGRID = E // EB
NB = 2000
NGRID = N // NB
FB = 400
FGRID = N // FB

PS1 = 64
PS2 = 16
PDP = 16


def _gelu(x):
    return 0.5 * x * (1.0 + lax.erf(x * 0.7071067811865476))


def _ln(x, g, b, eps=1e-5):
    mu = jnp.mean(x, axis=-1, keepdims=True)
    var = jnp.var(x, axis=-1, keepdims=True)
    return (x - mu) / jnp.sqrt(var + eps) * g + b


_RED16 = None


def _ln16(x, g, b, eps=1e-5):
    red = jnp.full((EH, EH), 1.0 / EH, jnp.float32)
    mu = x @ red
    var = (x * x) @ red - mu * mu
    return (x - mu) * lax.rsqrt(var + eps) * g + b


def _full(x):
    return pl.BlockSpec(x.shape, lambda *_: (0,) * x.ndim)


def _sc_gather(p_src1, p_src2, p_dst, src2, dst2):
    mesh = plsc.VectorSubcoreMesh(core_axis_name="c", subcore_axis_name="s",
                                  num_cores=NC, num_subcores=NS)

    @functools.partial(
        pl.kernel,
        out_type=jax.ShapeDtypeStruct((E, H), jnp.int32),
        mesh=mesh,
        scratch_types=[
            pltpu.VMEM((EPW,), jnp.int32),
            pltpu.VMEM((EPW,), jnp.int32),
            pltpu.VMEM((GNB, CH, PS1), jnp.int32),
            pltpu.VMEM((GNB, CH, PS2), jnp.int32),
            pltpu.VMEM((GNB, CH, PDP), jnp.int32),
            [pltpu.SemaphoreType.DMA] * GNB,
            [pltpu.SemaphoreType.DMA] * GNB,
            [pltpu.SemaphoreType.DMA] * GNB,
        ],
        compiler_params=pltpu.CompilerParams(use_tc_tiling_on_sc=False),
    )
    def k(p1_hbm, p2_hbm, pd_hbm, src_hbm, dst_hbm, g_out,
          sidx, didx, r1, r2, rd, sems1, sems2, semsd):
        wid = lax.axis_index("s") * NC + lax.axis_index("c")
        base = wid * EPW
        pltpu.sync_copy(src_hbm.at[wid], sidx)
        pltpu.sync_copy(dst_hbm.at[wid], didx)

        def body(q, carry):
            offs = [q * (GNB * CH) + kk * CH for kk in range(GNB)]
            copies = []
            for kk in range(GNB):
                sl = sidx.at[pl.ds(offs[kk], CH)]
                dl = didx.at[pl.ds(offs[kk], CH)]
                copies.append((
                    pltpu.async_copy(p1_hbm.at[sl], r1.at[kk], sems1[kk]),
                    pltpu.async_copy(p2_hbm.at[sl], r2.at[kk], sems2[kk]),
                    pltpu.async_copy(pd_hbm.at[dl], rd.at[kk], semsd[kk]),
                ))
            for kk in range(GNB):
                c1, c2, cd = copies[kk]
                rows = g_out.at[pl.ds(base + offs[kk], CH)]
                c1.wait()
                pltpu.sync_copy(r1.at[kk], rows.at[:, pl.ds(0, PS1)])
                c2.wait()
                pltpu.sync_copy(r2.at[kk], rows.at[:, pl.ds(PS1, PS2)])
                cd.wait()
                pltpu.sync_copy(rd.at[kk], rows.at[:, pl.ds(PS1 + PS2, PDP)])
            return carry

        lax.fori_loop(0, NCH // GNB, body, 0)

    return k(p_src1, p_src2, p_dst, src2, dst2)


def _pack(x, half):
    hi = x[:, :half].astype(jnp.bfloat16).astype(jnp.float32)
    lo = x[:, half:].astype(jnp.bfloat16).astype(jnp.float32)
    hi_w = lax.bitcast_convert_type(hi, jnp.int32)
    lo_w = lax.shift_right_logical(lax.bitcast_convert_type(lo, jnp.int32),
                                   16)
    return lax.bitwise_or(hi_w, lo_w)


def _unpack(w):
    hi = lax.bitcast_convert_type(
        lax.bitwise_and(w, jnp.int32(-65536)), jnp.float32)
    lo = lax.bitcast_convert_type(lax.shift_left(w, 16), jnp.float32)
    return hi, lo


def _proj_body(h_ref, wp1, bp1, wp2, wpd, p1_out, p2_out, pd_out):
    h = h_ref[...]
    m1 = h @ wp1[...] + bp1[...]
    p2 = h @ wp2[...]
    pd = h @ wpd[...]
    p1_out[...] = _pack(m1, PS1)
    p2_out[...] = _pack(p2, PS2)
    pd_out[...] = _pack(pd, PDP)


def _proj(h, wp1, bp1, wp2, wpd):
    return pl.pallas_call(
        _proj_body,
        grid=(NGRID,),
        in_specs=[pl.BlockSpec((NB, H), lambda i: (i, 0)),
                  _full(wp1), _full(bp1), _full(wp2), _full(wpd)],
        out_specs=(pl.BlockSpec((NB, PS1), lambda i: (i, 0)),
                   pl.BlockSpec((NB, PS2), lambda i: (i, 0)),
                   pl.BlockSpec((NB, PDP), lambda i: (i, 0))),
        out_shape=(jax.ShapeDtypeStruct((N, PS1), jnp.int32),
                   jax.ShapeDtypeStruct((N, PS2), jnp.int32),
                   jax.ShapeDtypeStruct((N, PDP), jnp.int32)),
    )(h, wp1, bp1, wp2, wpd)


def _sc_scatter(m, dst3, zeros_nh):
    mesh = plsc.VectorSubcoreMesh(core_axis_name="c", subcore_axis_name="s",
                                  num_cores=NC, num_subcores=NS)

    @functools.partial(
        pl.kernel,
        out_type=jax.ShapeDtypeStruct((2 * N, H), jnp.float32),
        mesh=mesh,
        scratch_types=[
            pltpu.VMEM((NCH, CH), jnp.int32),
            pltpu.VMEM((2, CH, H), jnp.float32),
            pltpu.VMEM_SHARED((N, H), jnp.float32),
            pltpu.SemaphoreType.DMA,
            pltpu.SemaphoreType.DMA,
        ],
    )
    def k(m_hbm, dst_hbm, zeros_hbm, out_hbm, idxs, rows, shared,
          lsem0, lsem1):
        cid = lax.axis_index("c")
        sid = lax.axis_index("s")
        wid = sid * NC + cid
        base = wid * EPW
        row0 = sid * RPT

        @pl.when(sid < NS - 1)
        def _():
            pltpu.sync_copy(zeros_hbm.at[pl.ds(row0, RPT)],
                            shared.at[pl.ds(row0, RPT)])

        @pl.when(sid == NS - 1)
        def _():
            pltpu.sync_copy(zeros_hbm.at[pl.ds((NS - 1) * RPT, RPT_LAST)],
                            shared.at[pl.ds((NS - 1) * RPT, RPT_LAST)])

        pltpu.sync_copy(dst_hbm.at[wid], idxs)
        plsc.subcore_barrier()

        def body(q, carry):
            c0 = q * 2
            l0 = pltpu.async_copy(m_hbm.at[pl.ds(base + c0 * CH, CH)],
                                  rows.at[0], lsem0)
            l1 = pltpu.async_copy(m_hbm.at[pl.ds(base + (c0 + 1) * CH, CH)],
                                  rows.at[1], lsem1)
            l0.wait()
            pltpu.sync_copy(rows.at[0], shared.at[idxs.at[c0]], add=True)
            l1.wait()
            pltpu.sync_copy(rows.at[1], shared.at[idxs.at[c0 + 1]], add=True)
            return carry

        lax.fori_loop(0, NCH // 2, body, 0)
        fin = pltpu.async_copy(m_hbm.at[pl.ds(base + (NCH - 1) * CH, CH)],
                               rows.at[0], lsem0)
        fin.wait()
        pltpu.sync_copy(rows.at[0], shared.at[idxs.at[NCH - 1]], add=True)
        plsc.subcore_barrier()

        obase = cid * N + row0

        @pl.when(sid < NS - 1)
        def _():
            pltpu.sync_copy(shared.at[pl.ds(row0, RPT)],
                            out_hbm.at[pl.ds(obase, RPT)])

        @pl.when(sid == NS - 1)
        def _():
            pltpu.sync_copy(shared.at[pl.ds((NS - 1) * RPT, RPT_LAST)],
                            out_hbm.at[pl.ds(cid * N + (NS - 1) * RPT,
                                             RPT_LAST)])

    return k(m, dst3, zeros_nh)


def _make_edge_body(first):
    def body(g_ref, e_ref, wqe, bq, emw2, emb2,
             eng, enb, w1e, w2, b2, ein_w, ein_b, ein_g, ein_bb,
             e_out, m_out):
        gall = g_ref[...]
        m1_lo, m1_hi = _unpack(gall[:, :PS1])
        sa, sb = _unpack(gall[:, PS1:PS1 + PS2])
        da, db = _unpack(gall[:, PS1 + PS2:PS1 + PS2 + PDP])
        if first:
            x = e_ref[...] @ ein_w[...] + ein_b[...]
            e = _ln16(_gelu(x), ein_g[...], ein_bb[...])
        else:
            e = e_ref[...]
        qe = e @ wqe[...] + bq[...]
        glogit = sb[:, 0:1] + db[:, 0:1] + qe[:, EH:EH + 1]
        gate = jax.nn.sigmoid(glogit)
        dpre = sa + da + qe[:, :EH]
        delta = _gelu(dpre) @ emw2[...] + emb2[...]
        e2 = _ln16(e + EDGE_SCALE * delta * gate, eng[...], enb[...])
        m1 = jnp.concatenate([m1_lo, m1_hi], axis=1)
        t = _gelu(m1 + e2 @ w1e[...])
        m_out[...] = t @ w2[...] + b2[...]
        e_out[...] = e2
    return body


def _edge_layer(gall, e, weights, first):
    wspecs = [_full(w) for w in weights]
    return pl.pallas_call(
        _make_edge_body(first),
        grid=(EGRID,),
        in_specs=[pl.BlockSpec((EB, H), lambda i: (i, 0)),
                  pl.BlockSpec((EB, EH), lambda i: (i, 0))] + wspecs,
        out_specs=(pl.BlockSpec((EB, EH), lambda i: (i, 0)),
                   pl.BlockSpec((EB, H), lambda i: (i, 0))),
        out_shape=(jax.ShapeDtypeStruct((E, EH), jnp.float32),
                   jax.ShapeDtypeStruct((E, H), jnp.float32)),
    )(gall, e, *weights)


def _node_update_body(h_ref, a0_ref, a1_ref, w1h, w1a, b1, w2, b2, ng, nb,
                      out_ref):
    h = h_ref[...]
    agg = a0_ref[...] + a1_ref[...]
    u = _gelu(h @ w1h[...] + agg @ w1a[...] + b1[...]) @ w2[...] + b2[...]
    out_ref[...] = _ln(h + u, ng[...], nb[...])


def _node_update(h, a0, a1, weights):
    wspecs = [_full(w) for w in weights]
    return pl.pallas_call(
        _node_update_body,
        grid=(NGRID,),
        in_specs=[pl.BlockSpec((NB, H), lambda i: (i, 0)),
                  pl.BlockSpec((NB, H), lambda i: (i, 0)),
                  pl.BlockSpec((NB, H), lambda i: (i, 0))] + wspecs,
        out_specs=pl.BlockSpec((NB, H), lambda i: (i, 0)),
        out_shape=jax.ShapeDtypeStruct((N, H), jnp.float32),
    )(h, a0, a1, *weights)


def _final_body(h_ref, b_ref, og, ob, h_out, mix_out, summ, cnt):
    i = pl.program_id(0)
    hn = _ln(h_ref[...], og[...], ob[...])
    h_out[...] = hn

    @pl.when(i == 0)
    def _():
        summ[...] = jnp.zeros_like(summ)
        cnt[...] = jnp.zeros_like(cnt)

    bids = b_ref[0, 0, :]
    gid = lax.broadcasted_iota(jnp.int32, (G, FB), 0)
    S = (gid == bids[None, :]).astype(jnp.float32)
    summ[...] += jnp.dot(S, hn)
    cnt[...] += jnp.dot(S, jnp.ones((FB, H), jnp.float32))

    @pl.when(i == FGRID - 1)
    def _():
        mix_out[...] = summ[...] / jnp.clip(cnt[...], 1.0, None)


def _final_pool(h, batch3, og, ob):
    return pl.pallas_call(
        _final_body,
        grid=(FGRID,),
        in_specs=[pl.BlockSpec((FB, H), lambda i: (i, 0)),
                  pl.BlockSpec((1, 1, FB), lambda i: (i, 0, 0)),
                  _full(og), _full(ob)],
        out_specs=(pl.BlockSpec((FB, H), lambda i: (i, 0)),
                   pl.BlockSpec((G, H), lambda i: (0, 0))),
        out_shape=(jax.ShapeDtypeStruct((N, H), jnp.float32),
                   jax.ShapeDtypeStruct((G, H), jnp.float32)),
        scratch_shapes=[pltpu.VMEM((G, H), jnp.float32),
                        pltpu.VMEM((G, H), jnp.float32)],
    )(h, batch3, og, ob)


def _row(x):
    return x.reshape(1, -1).astype(jnp.float32)


def kernel(node_h, edge_index, edge_attr, batch, fallback_num_graphs, params):
    src = edge_index[0].astype(jnp.int32)
    dst = edge_index[1].astype(jnp.int32)
    src2 = src.reshape(NW, EPW)
    dst2 = dst.reshape(NW, EPW)
    dst3 = dst.reshape(NW, NCH, CH)
    batch3 = batch.astype(jnp.int32).reshape(FGRID, 1, FB)
    zeros_nh = jnp.zeros((N, H), jnp.float32)

    e = edge_attr
    ein = [params['edge_in_W'], _row(params['edge_in_b']),
           _row(params['edge_norm_g']), _row(params['edge_norm_b'])]
    h = node_h
    for li, lp in enumerate(params['layers']):
        em_W1 = lp['em_W1']
        eg_W = lp['eg_W']
        wp1 = lp['msg_W1'][:H]
        bp1 = _row(lp['msg_b1'])
        wp2 = jnp.zeros((H, 2 * PS2), jnp.float32)
        wp2 = wp2.at[:, :EH].set(em_W1[:H])
        wp2 = wp2.at[:, EH].set(eg_W[:H, 0])
        wpd = jnp.zeros((H, 2 * PDP), jnp.float32)
        wpd = wpd.at[:, :EH].set(em_W1[H:2 * H])
        wpd = wpd.at[:, EH].set(eg_W[H:2 * H, 0])
        wqe = jnp.zeros((EH, 32), jnp.float32)
        wqe = wqe.at[:, :EH].set(em_W1[2 * H:]).at[:, EH].set(eg_W[2 * H:, 0])
        bq = jnp.zeros((1, 32), jnp.float32)
        bq = bq.at[0, :EH].set(lp['em_b1']).at[0, EH].set(lp['eg_b'][0])

        edge_w = [wqe, bq, lp['em_W2'], _row(lp['em_b2']),
                  _row(lp['enorm_g']), _row(lp['enorm_b']),
                  lp['msg_W1'][H:], lp['msg_W2'], _row(lp['msg_b2'])] + ein
        upd_w = [lp['upd_W1'][:H], lp['upd_W1'][H:], _row(lp['upd_b1']),
                 lp['upd_W2'], _row(lp['upd_b2']),
                 _row(lp['norm_g']), _row(lp['norm_b'])]

        p1, p2, pd = _proj(h, wp1, bp1, wp2, wpd)
        gall = _sc_gather(p1, p2, pd, src2, dst2)
        e, m = _edge_layer(gall, e, edge_w, li == 0)
        parts = _sc_scatter(m, dst3, zeros_nh)
        h = _node_update(h, parts[:N], parts[N:], upd_w)

    h_out, mix = _final_pool(h, batch3, _row(params['out_norm_g']),
                             _row(params['out_norm_b']))
    scale = fallback_num_graphs.astype(jnp.float32) / jnp.float32(G) \
        if hasattr(fallback_num_graphs, 'astype') \
        else jnp.float32(fallback_num_graphs) / jnp.float32(G)
    mix = mix * scale
    return h_out, mix


# speedup vs baseline: 1.6927x; 1.0273x over previous
import functools

import jax
import jax.numpy as jnp
from jax import lax
from jax.experimental import pallas as pl
from jax.experimental.pallas import tpu as pltpu
from jax.experimental.pallas import tpu_sc as plsc

H = 128
EH = 16
E = 320000
N = 10000
G = 2500
EDGE_SCALE = 0.1

NC = 2
NS = 16
NW = NC * NS
EPW = E // NW
CH = 80
NCH = EPW // CH
GNB = 5
RPT = 632
RPT_LAST = N - (NS - 1) * RPT

EB = 2560
EGRID = E // EB
NB = 2000
NGRID = N // NB
FB = 400
FGRID = N // FB

PS1 = 64
PS2 = 16
PDP = 16


def _gelu(x):
    return 0.5 * x * (1.0 + lax.erf(x * 0.7071067811865476))


def _ln(x, g, b, eps=1e-5):
    mu = jnp.mean(x, axis=-1, keepdims=True)
    var = jnp.var(x, axis=-1, keepdims=True)
    return (x - mu) / jnp.sqrt(var + eps) * g + b


_RED16 = None


def _ln16(x, g, b, eps=1e-5):
    red = jnp.full((EH, EH), 1.0 / EH, jnp.float32)
    mu = x @ red
    var = (x * x) @ red - mu * mu
    return (x - mu) * lax.rsqrt(var + eps) * g + b


def _full(x):
    return pl.BlockSpec(x.shape, lambda *_: (0,) * x.ndim)


def _sc_gather(p_src1, p_src2, p_dst, src2, dst2, e_tot, epw, ch):
    nch = epw // ch
    mesh = plsc.VectorSubcoreMesh(core_axis_name="c", subcore_axis_name="s",
                                  num_cores=NC, num_subcores=NS)

    @functools.partial(
        pl.kernel,
        out_type=jax.ShapeDtypeStruct((e_tot, H), jnp.int32),
        mesh=mesh,
        scratch_types=[
            pltpu.VMEM((epw,), jnp.int32),
            pltpu.VMEM((epw,), jnp.int32),
            pltpu.VMEM((GNB, ch, PS1), jnp.int32),
            pltpu.VMEM((GNB, ch, PS2), jnp.int32),
            pltpu.VMEM((GNB, ch, PDP), jnp.int32),
            [pltpu.SemaphoreType.DMA] * GNB,
            [pltpu.SemaphoreType.DMA] * GNB,
            [pltpu.SemaphoreType.DMA] * GNB,
        ],
        compiler_params=pltpu.CompilerParams(use_tc_tiling_on_sc=False),
    )
    def k(p1_hbm, p2_hbm, pd_hbm, src_hbm, dst_hbm, g_out,
          sidx, didx, r1, r2, rd, sems1, sems2, semsd):
        wid = lax.axis_index("s") * NC + lax.axis_index("c")
        base = wid * epw
        pltpu.sync_copy(src_hbm.at[wid], sidx)
        pltpu.sync_copy(dst_hbm.at[wid], didx)

        def body(q, carry):
            offs = [q * (GNB * ch) + kk * ch for kk in range(GNB)]
            copies = []
            for kk in range(GNB):
                sl = sidx.at[pl.ds(offs[kk], ch)]
                dl = didx.at[pl.ds(offs[kk], ch)]
                copies.append((
                    pltpu.async_copy(p1_hbm.at[sl], r1.at[kk], sems1[kk]),
                    pltpu.async_copy(p2_hbm.at[sl], r2.at[kk], sems2[kk]),
                    pltpu.async_copy(pd_hbm.at[dl], rd.at[kk], semsd[kk]),
                ))
            for kk in range(GNB):
                c1, c2, cd = copies[kk]
                rows = g_out.at[pl.ds(base + offs[kk], ch)]
                c1.wait()
                pltpu.sync_copy(r1.at[kk], rows.at[:, pl.ds(0, PS1)])
                c2.wait()
                pltpu.sync_copy(r2.at[kk], rows.at[:, pl.ds(PS1, PS2)])
                cd.wait()
                pltpu.sync_copy(rd.at[kk], rows.at[:, pl.ds(PS1 + PS2, PDP)])
            return carry

        lax.fori_loop(0, nch // GNB, body, 0)

    return k(p_src1, p_src2, p_dst, src2, dst2)


def _pack(x, half):
    hi = x[:, :half].astype(jnp.bfloat16).astype(jnp.float32)
    lo = x[:, half:].astype(jnp.bfloat16).astype(jnp.float32)
    hi_w = lax.bitcast_convert_type(hi, jnp.int32)
    lo_w = lax.shift_right_logical(lax.bitcast_convert_type(lo, jnp.int32),
                                   16)
    return lax.bitwise_or(hi_w, lo_w)


def _unpack(w):
    hi = lax.bitcast_convert_type(
        lax.bitwise_and(w, jnp.int32(-65536)), jnp.float32)
    lo = lax.bitcast_convert_type(lax.shift_left(w, 16), jnp.float32)
    return hi, lo


def _proj_body(h_ref, wp1, bp1, wp2, wpd, p1_out, p2_out, pd_out):
    h = h_ref[...]
    m1 = h @ wp1[...] + bp1[...]
    p2 = h @ wp2[...]
    pd = h @ wpd[...]
    p1_out[...] = _pack(m1, PS1)
    p2_out[...] = _pack(p2, PS2)
    pd_out[...] = _pack(pd, PDP)


def _proj(h, wp1, bp1, wp2, wpd):
    return pl.pallas_call(
        _proj_body,
        grid=(NGRID,),
        in_specs=[pl.BlockSpec((NB, H), lambda i: (i, 0)),
                  _full(wp1), _full(bp1), _full(wp2), _full(wpd)],
        out_specs=(pl.BlockSpec((NB, PS1), lambda i: (i, 0)),
                   pl.BlockSpec((NB, PS2), lambda i: (i, 0)),
                   pl.BlockSpec((NB, PDP), lambda i: (i, 0))),
        out_shape=(jax.ShapeDtypeStruct((N, PS1), jnp.int32),
                   jax.ShapeDtypeStruct((N, PS2), jnp.int32),
                   jax.ShapeDtypeStruct((N, PDP), jnp.int32)),
    )(h, wp1, bp1, wp2, wpd)


def _sc_scatter(m, dst3, zeros_nh, epw, ch):
    nch = epw // ch
    mesh = plsc.VectorSubcoreMesh(core_axis_name="c", subcore_axis_name="s",
                                  num_cores=NC, num_subcores=NS)

    @functools.partial(
        pl.kernel,
        out_type=jax.ShapeDtypeStruct((2 * N, H), jnp.float32),
        mesh=mesh,
        scratch_types=[
            pltpu.VMEM((nch, ch), jnp.int32),
            pltpu.VMEM((2, ch, H), jnp.float32),
            pltpu.VMEM_SHARED((N, H), jnp.float32),
            pltpu.SemaphoreType.DMA,
            pltpu.SemaphoreType.DMA,
        ],
    )
    def k(m_hbm, dst_hbm, zeros_hbm, out_hbm, idxs, rows, shared,
          lsem0, lsem1):
        cid = lax.axis_index("c")
        sid = lax.axis_index("s")
        wid = sid * NC + cid
        base = wid * epw
        row0 = sid * RPT

        @pl.when(sid < NS - 1)
        def _():
            pltpu.sync_copy(zeros_hbm.at[pl.ds(row0, RPT)],
                            shared.at[pl.ds(row0, RPT)])

        @pl.when(sid == NS - 1)
        def _():
            pltpu.sync_copy(zeros_hbm.at[pl.ds((NS - 1) * RPT, RPT_LAST)],
                            shared.at[pl.ds((NS - 1) * RPT, RPT_LAST)])

        pltpu.sync_copy(dst_hbm.at[wid], idxs)
        plsc.subcore_barrier()

        def body(q, carry):
            c0 = q * 2
            l0 = pltpu.async_copy(m_hbm.at[pl.ds(base + c0 * ch, ch)],
                                  rows.at[0], lsem0)
            l1 = pltpu.async_copy(m_hbm.at[pl.ds(base + (c0 + 1) * ch, ch)],
                                  rows.at[1], lsem1)
            l0.wait()
            pltpu.sync_copy(rows.at[0], shared.at[idxs.at[c0]], add=True)
            l1.wait()
            pltpu.sync_copy(rows.at[1], shared.at[idxs.at[c0 + 1]], add=True)
            return carry

        lax.fori_loop(0, nch // 2, body, 0)
        if nch % 2:
            fin = pltpu.async_copy(m_hbm.at[pl.ds(base + (nch - 1) * ch, ch)],
                                   rows.at[0], lsem0)
            fin.wait()
            pltpu.sync_copy(rows.at[0], shared.at[idxs.at[nch - 1]], add=True)
        plsc.subcore_barrier()

        obase = cid * N + row0

        @pl.when(sid < NS - 1)
        def _():
            pltpu.sync_copy(shared.at[pl.ds(row0, RPT)],
                            out_hbm.at[pl.ds(obase, RPT)])

        @pl.when(sid == NS - 1)
        def _():
            pltpu.sync_copy(shared.at[pl.ds((NS - 1) * RPT, RPT_LAST)],
                            out_hbm.at[pl.ds(cid * N + (NS - 1) * RPT,
                                             RPT_LAST)])

    return k(m, dst3, zeros_nh)


def _make_edge_body(first):
    def body(g_ref, e_ref, wqe, bq, emw2, emb2,
             eng, enb, w1e, w2, b2, ein_w, ein_b, ein_g, ein_bb,
             e_out, m_out):
        gall = g_ref[...]
        m1_lo, m1_hi = _unpack(gall[:, :PS1])
        sa, sb = _unpack(gall[:, PS1:PS1 + PS2])
        da, db = _unpack(gall[:, PS1 + PS2:PS1 + PS2 + PDP])
        if first:
            x = e_ref[...] @ ein_w[...] + ein_b[...]
            e = _ln16(_gelu(x), ein_g[...], ein_bb[...])
        else:
            e = e_ref[...]
        qe = e @ wqe[...] + bq[...]
        glogit = sb[:, 0:1] + db[:, 0:1] + qe[:, EH:EH + 1]
        gate = jax.nn.sigmoid(glogit)
        dpre = sa + da + qe[:, :EH]
        delta = _gelu(dpre) @ emw2[...] + emb2[...]
        e2 = _ln16(e + EDGE_SCALE * delta * gate, eng[...], enb[...])
        m1 = jnp.concatenate([m1_lo, m1_hi], axis=1)
        t = _gelu(m1 + e2 @ w1e[...])
        m_out[...] = t @ w2[...] + b2[...]
        e_out[...] = e2
    return body


def _edge_layer(gall, e, weights, first):
    e_tot = gall.shape[0]
    eb = 2000
    wspecs = [_full(w) for w in weights]
    return pl.pallas_call(
        _make_edge_body(first),
        grid=(e_tot // eb,),
        in_specs=[pl.BlockSpec((eb, H), lambda i: (i, 0)),
                  pl.BlockSpec((eb, EH), lambda i: (i, 0))] + wspecs,
        out_specs=(pl.BlockSpec((eb, EH), lambda i: (i, 0)),
                   pl.BlockSpec((eb, H), lambda i: (i, 0))),
        out_shape=(jax.ShapeDtypeStruct((e_tot, EH), jnp.float32),
                   jax.ShapeDtypeStruct((e_tot, H), jnp.float32)),
    )(gall, e, *weights)


def _node_update_body(h_ref, a0_ref, a1_ref, a2_ref, a3_ref,
                      w1h, w1a, b1, w2, b2, ng, nb, out_ref):
    h = h_ref[...]
    agg = (a0_ref[...] + a1_ref[...]) + (a2_ref[...] + a3_ref[...])
    u = _gelu(h @ w1h[...] + agg @ w1a[...] + b1[...]) @ w2[...] + b2[...]
    out_ref[...] = _ln(h + u, ng[...], nb[...])


def _node_update(h, a0, a1, a2, a3, weights):
    wspecs = [_full(w) for w in weights]
    return pl.pallas_call(
        _node_update_body,
        grid=(NGRID,),
        in_specs=[pl.BlockSpec((NB, H), lambda i: (i, 0)),
                  pl.BlockSpec((NB, H), lambda i: (i, 0)),
                  pl.BlockSpec((NB, H), lambda i: (i, 0)),
                  pl.BlockSpec((NB, H), lambda i: (i, 0)),
                  pl.BlockSpec((NB, H), lambda i: (i, 0))] + wspecs,
        out_specs=pl.BlockSpec((NB, H), lambda i: (i, 0)),
        out_shape=jax.ShapeDtypeStruct((N, H), jnp.float32),
    )(h, a0, a1, a2, a3, *weights)


def _final_body(h_ref, b_ref, og, ob, h_out, mix_out, summ, cnt):
    i = pl.program_id(0)
    hn = _ln(h_ref[...], og[...], ob[...])
    h_out[...] = hn

    @pl.when(i == 0)
    def _():
        summ[...] = jnp.zeros_like(summ)
        cnt[...] = jnp.zeros_like(cnt)

    bids = b_ref[0, 0, :]
    gid = lax.broadcasted_iota(jnp.int32, (G, FB), 0)
    S = (gid == bids[None, :]).astype(jnp.float32)
    summ[...] += jnp.dot(S, hn)
    cnt[...] += jnp.dot(S, jnp.ones((FB, H), jnp.float32))

    @pl.when(i == FGRID - 1)
    def _():
        mix_out[...] = summ[...] / jnp.clip(cnt[...], 1.0, None)


def _final_pool(h, batch3, og, ob):
    return pl.pallas_call(
        _final_body,
        grid=(FGRID,),
        in_specs=[pl.BlockSpec((FB, H), lambda i: (i, 0)),
                  pl.BlockSpec((1, 1, FB), lambda i: (i, 0, 0)),
                  _full(og), _full(ob)],
        out_specs=(pl.BlockSpec((FB, H), lambda i: (i, 0)),
                   pl.BlockSpec((G, H), lambda i: (0, 0))),
        out_shape=(jax.ShapeDtypeStruct((N, H), jnp.float32),
                   jax.ShapeDtypeStruct((G, H), jnp.float32)),
        scratch_shapes=[pltpu.VMEM((G, H), jnp.float32),
                        pltpu.VMEM((G, H), jnp.float32)],
    )(h, batch3, og, ob)


def _row(x):
    return x.reshape(1, -1).astype(jnp.float32)


def kernel(node_h, edge_index, edge_attr, batch, fallback_num_graphs, params):
    src = edge_index[0].astype(jnp.int32)
    dst = edge_index[1].astype(jnp.int32)
    EHALF = E // 2
    EPW2 = EHALF // NW
    CH2 = 40
    srch = [src[:EHALF].reshape(NW, EPW2), src[EHALF:].reshape(NW, EPW2)]
    dsth = [dst[:EHALF].reshape(NW, EPW2), dst[EHALF:].reshape(NW, EPW2)]
    dsth3 = [dst[:EHALF].reshape(NW, EPW2 // CH2, CH2),
             dst[EHALF:].reshape(NW, EPW2 // CH2, CH2)]
    batch3 = batch.astype(jnp.int32).reshape(FGRID, 1, FB)
    zeros_nh = jnp.zeros((N, H), jnp.float32)

    eh = [edge_attr[:E // 2], edge_attr[E // 2:]]
    ein = [params['edge_in_W'], _row(params['edge_in_b']),
           _row(params['edge_norm_g']), _row(params['edge_norm_b'])]
    h = node_h
    for li, lp in enumerate(params['layers']):
        em_W1 = lp['em_W1']
        eg_W = lp['eg_W']
        wp1 = lp['msg_W1'][:H]
        bp1 = _row(lp['msg_b1'])
        wp2 = jnp.zeros((H, 2 * PS2), jnp.float32)
        wp2 = wp2.at[:, :EH].set(em_W1[:H])
        wp2 = wp2.at[:, EH].set(eg_W[:H, 0])
        wpd = jnp.zeros((H, 2 * PDP), jnp.float32)
        wpd = wpd.at[:, :EH].set(em_W1[H:2 * H])
        wpd = wpd.at[:, EH].set(eg_W[H:2 * H, 0])
        wqe = jnp.zeros((EH, 32), jnp.float32)
        wqe = wqe.at[:, :EH].set(em_W1[2 * H:]).at[:, EH].set(eg_W[2 * H:, 0])
        bq = jnp.zeros((1, 32), jnp.float32)
        bq = bq.at[0, :EH].set(lp['em_b1']).at[0, EH].set(lp['eg_b'][0])

        edge_w = [wqe, bq, lp['em_W2'], _row(lp['em_b2']),
                  _row(lp['enorm_g']), _row(lp['enorm_b']),
                  lp['msg_W1'][H:], lp['msg_W2'], _row(lp['msg_b2'])] + ein
        upd_w = [lp['upd_W1'][:H], lp['upd_W1'][H:], _row(lp['upd_b1']),
                 lp['upd_W2'], _row(lp['upd_b2']),
                 _row(lp['norm_g']), _row(lp['norm_b'])]

        p1, p2, pd = _proj(h, wp1, bp1, wp2, wpd)
        gA = _sc_gather(p1, p2, pd, srch[0], dsth[0], EHALF, EPW2, CH2)
        gB = _sc_gather(p1, p2, pd, srch[1], dsth[1], EHALF, EPW2, CH2)
        eA, mA = _edge_layer(gA, eh[0], edge_w, li == 0)
        eB, mB = _edge_layer(gB, eh[1], edge_w, li == 0)
        eh = [eA, eB]
        pA = _sc_scatter(mA, dsth3[0], zeros_nh, EPW2, CH2)
        pB = _sc_scatter(mB, dsth3[1], zeros_nh, EPW2, CH2)
        h = _node_update(h, pA[:N], pA[N:], pB[:N], pB[N:], upd_w)

    h_out, mix = _final_pool(h, batch3, _row(params['out_norm_g']),
                             _row(params['out_norm_b']))
    scale = fallback_num_graphs.astype(jnp.float32) / jnp.float32(G) \
        if hasattr(fallback_num_graphs, 'astype') \
        else jnp.float32(fallback_num_graphs) / jnp.float32(G)
    mix = mix * scale
    return h_out, mix
